# baseline (speedup 1.0000x reference)
import jax
import jax.numpy as jnp
from jax.experimental import pallas as pl

_N_BLK = 2048


def _mm_kernel(u_ref, i_ref, o_ref):
    o_ref[...] = jax.lax.dot_general(
        u_ref[...],
        i_ref[...],
        dimension_numbers=(((1,), (1,)), ((), ())),
        preferred_element_type=jnp.float32,
    )


def kernel(u_g_embeddings, i_g_embeddings):
    M, K = u_g_embeddings.shape
    N = i_g_embeddings.shape[0]
    return pl.pallas_call(
        _mm_kernel,
        grid=(pl.cdiv(N, _N_BLK),),
        in_specs=[
            pl.BlockSpec((M, K), lambda j: (0, 0)),
            pl.BlockSpec((_N_BLK, K), lambda j: (j, 0)),
        ],
        out_specs=pl.BlockSpec((M, _N_BLK), lambda j: (0, j)),
        out_shape=jax.ShapeDtypeStruct((M, N), jnp.float32),
    )(u_g_embeddings, i_g_embeddings)
